# Initial kernel scaffold; baseline (speedup 1.0000x reference)
import jax
import jax.numpy as jnp
from jax.experimental import pallas as pl


def kernel(author_embedding, paper_embedding, adj_values, W1, b1, W2, b2, edge_index):
    raise NotImplementedError("write your pallas kernel here")


# speedup vs baseline: 4.0804x; 4.0804x over previous
import functools

import jax
import jax.numpy as jnp
from jax import lax
from jax.experimental import pallas as pl
from jax.experimental.pallas import tpu as pltpu
from jax.experimental.pallas import tpu_sc as plsc

N_AUTHORS = 5000
N_PAPERS = 5000
N = N_AUTHORS + N_PAPERS
E = 320000
D = 128
L = 3

NC = 2
NS = 16
NW = NC * NS
EPW = E // NW
CHUNK = 80
NCHUNK = EPW // CHUNK
OBUF_ROWS = 80
NRCHUNK = N // OBUF_ROWS
RITER = -(-NRCHUNK // NS)


def _spmm_body(x_hbm, col_hbm, row_hbm, val_hbm, out_hbm,
               colv, rowv, valv, rows, obuf, acc, sem):
    cid = lax.axis_index("c")
    sid = lax.axis_index("s")
    wid = cid * NS + sid

    zv = jnp.zeros((16,), jnp.float32)

    def zero_row(i, carry):
        for db in range(D // 16):
            obuf[i, pl.ds(db * 16, 16)] = zv
        return carry

    lax.fori_loop(0, OBUF_ROWS, zero_row, 0)
    for kk in range(RITER):
        rc = sid + kk * NS

        @pl.when(rc < NRCHUNK)
        def _():
            pltpu.sync_copy(obuf, acc.at[pl.ds(rc * OBUF_ROWS, OBUF_ROWS)])
    plsc.subcore_barrier()

    def chunk_body(j, carry):
        base = pl.multiple_of(wid * EPW + j * CHUNK, 8)
        pltpu.sync_copy(col_hbm.at[pl.ds(base, CHUNK)], colv)
        pltpu.sync_copy(row_hbm.at[pl.ds(base, CHUNK)], rowv)
        pltpu.sync_copy(val_hbm.at[pl.ds(base, CHUNK)], valv)
        pltpu.async_copy(x_hbm.at[colv], rows, sem).wait()

        def group_body(g, c2):
            vals16 = valv[pl.ds(g * 16, 16)]
            for j in range(16):
                e = g * 16 + j
                v = vals16[j]
                for db in range(D // 16):
                    sl = pl.ds(db * 16, 16)
                    rows[e, sl] = rows[e, sl] * v
            return c2

        lax.fori_loop(0, CHUNK // 16, group_body, 0)
        pltpu.sync_copy(rows, acc.at[rowv], add=True)
        return carry

    lax.fori_loop(0, NCHUNK, chunk_body, 0)
    plsc.subcore_barrier()

    for kk in range(RITER):
        rc = sid + kk * NS

        @pl.when(rc < NRCHUNK)
        def _():
            r0 = rc * OBUF_ROWS
            pltpu.sync_copy(acc.at[pl.ds(r0, OBUF_ROWS)], obuf)
            pltpu.sync_copy(obuf, out_hbm.at[cid, pl.ds(r0, OBUF_ROWS)])


_spmm_sc = functools.partial(
    pl.kernel,
    mesh=plsc.VectorSubcoreMesh(core_axis_name="c", subcore_axis_name="s"),
    out_type=jax.ShapeDtypeStruct((NC, N, D), jnp.float32),
    scratch_types=[
        pltpu.VMEM((CHUNK,), jnp.int32),
        pltpu.VMEM((CHUNK,), jnp.int32),
        pltpu.VMEM((CHUNK,), jnp.float32),
        pltpu.VMEM((CHUNK, D), jnp.float32),
        pltpu.VMEM((OBUF_ROWS, D), jnp.float32),
        pltpu.VMEM_SHARED((N, D), jnp.float32),
        pltpu.SemaphoreType.DMA,
    ],
)(_spmm_body)


BLK = 1000


def _dense_body(part_ref, ego_ref, w1_ref, b1_ref, w2_ref, b2_ref,
                egon_ref, norm_ref):
    side = part_ref[0] + part_ref[1]
    ego = ego_ref[...]
    s1 = lax.dot_general(side, w1_ref[...], (((1,), (1,)), ((), ())),
                         preferred_element_type=jnp.float32) + b1_ref[...]
    s1 = jnp.where(s1 >= 0, s1, 0.01 * s1)
    s2 = lax.dot_general(ego * side, w2_ref[...], (((1,), (1,)), ((), ())),
                         preferred_element_type=jnp.float32) + b2_ref[...]
    s2 = jnp.where(s2 >= 0, s2, 0.01 * s2)
    e = s1 + s2
    egon_ref[...] = e
    nrm = jnp.sqrt(jnp.sum(e * e, axis=1, keepdims=True))
    norm_ref[...] = e / jnp.maximum(nrm, 1e-12)


_dense_tc = pl.pallas_call(
    _dense_body,
    grid=(N // BLK,),
    in_specs=[
        pl.BlockSpec((NC, BLK, D), lambda i: (0, i, 0)),
        pl.BlockSpec((BLK, D), lambda i: (i, 0)),
        pl.BlockSpec((D, D), lambda i: (0, 0)),
        pl.BlockSpec((1, D), lambda i: (0, 0)),
        pl.BlockSpec((D, D), lambda i: (0, 0)),
        pl.BlockSpec((1, D), lambda i: (0, 0)),
    ],
    out_specs=[
        pl.BlockSpec((BLK, D), lambda i: (i, 0)),
        pl.BlockSpec((BLK, D), lambda i: (i, 0)),
    ],
    out_shape=[
        jax.ShapeDtypeStruct((N, D), jnp.float32),
        jax.ShapeDtypeStruct((N, D), jnp.float32),
    ],
)


def kernel(author_embedding, paper_embedding, adj_values, W1, b1, W2, b2,
           edge_index):
    ego = jnp.concatenate([author_embedding, paper_embedding], axis=0)
    row = edge_index[0]
    col = edge_index[1]
    outs = [ego]
    for k in range(L):
        part = _spmm_sc(ego, col, row, adj_values)
        ego, nrm = _dense_tc(part, ego, W1[k], b1[k].reshape(1, D),
                             W2[k], b2[k].reshape(1, D))
        outs.append(nrm)
    all_emb = jnp.concatenate(outs, axis=1)
    return (all_emb[:N_AUTHORS], all_emb[N_AUTHORS:])
